# Initial kernel scaffold; baseline (speedup 1.0000x reference)
import jax
import jax.numpy as jnp
from jax.experimental import pallas as pl


def kernel(vertices, faces):
    raise NotImplementedError("write your pallas kernel here")


# speedup vs baseline: 22.5402x; 22.5402x over previous
import functools

import jax
import jax.numpy as jnp
from jax import lax
from jax.experimental import pallas as pl
from jax.experimental.pallas import tpu as pltpu
from jax.experimental.pallas import tpu_sc as plsc

NC = 2
NS = 16
NW = NC * NS
L = 16

V = 100000
F = 200000
VP = 102400
FW = 6400
G = 128
GROUPS = FW // G
VS = VP // NS
VN = VP // NW


def _accumulate_body(verts_hbm, idx_hbm, zeros_hbm, part_hbm,
                     vx, vy, vz, ax, ay, az,
                     i0, i1, i2,
                     c0x, c0y, c0z, c1x, c1y, c1z, c2x, c2y, c2z,
                     nx, ny, nz):
    c = lax.axis_index("c")
    s = lax.axis_index("s")
    wid = c * NS + s
    vtab = (vx, vy, vz)
    atab = (ax, ay, az)
    itab = (i0, i1, i2)
    ctab = ((c0x, c0y, c0z), (c1x, c1y, c1z), (c2x, c2y, c2z))
    ntab = (nx, ny, nz)

    sl = pl.ds(s * VS, VS)
    for k in range(3):
        pltpu.sync_copy(verts_hbm.at[pl.ds(k * VP + s * VS, VS)], vtab[k].at[sl])
        pltpu.sync_copy(zeros_hbm.at[sl], atab[k].at[sl])
    plsc.subcore_barrier()

    ibase = wid * 3 * GROUPS * G

    def group(g, _):
        for corner in range(3):
            off = ibase + (corner * GROUPS) * G + g * G
            pltpu.sync_copy(idx_hbm.at[pl.ds(off, G)], itab[corner])
            for k in range(3):
                pltpu.sync_copy(vtab[k].at[itab[corner]], ctab[corner][k])
        for t in range(G // L):
            ts = pl.ds(t * L, L)
            x0, y0, z0 = c0x[ts], c0y[ts], c0z[ts]
            x1, y1, z1 = c1x[ts], c1y[ts], c1z[ts]
            x2, y2, z2 = c2x[ts], c2y[ts], c2z[ts]
            e1x, e1y, e1z = x1 - x0, y1 - y0, z1 - z0
            e2x, e2y, e2z = x2 - x0, y2 - y0, z2 - z0
            nx[ts] = e1y * e2z - e1z * e2y
            ny[ts] = e1z * e2x - e1x * e2z
            nz[ts] = e1x * e2y - e1y * e2x
        for corner in range(3):
            for k in range(3):
                pltpu.sync_copy(ntab[k], atab[k].at[itab[corner]], add=True)
        return ()

    lax.fori_loop(0, GROUPS, group, (), unroll=False)
    plsc.subcore_barrier()
    for k in range(3):
        pltpu.sync_copy(atab[k].at[sl],
                        part_hbm.at[pl.ds((c * 3 + k) * VP + s * VS, VS)])


def _normalize_body(part_hbm, out_hbm, pax, pay, paz, pbx, pby, pbz,
                    ox, oy, oz):
    c = lax.axis_index("c")
    s = lax.axis_index("s")
    wid = c * NS + s
    base = wid * VN
    ptab = ((pax, pay, paz), (pbx, pby, pbz))
    for sc in range(2):
        for k in range(3):
            pltpu.sync_copy(part_hbm.at[pl.ds((sc * 3 + k) * VP + base, VN)],
                            ptab[sc][k])

    def step(t, _):
        ts = pl.ds(t * L, L)
        x = pax[ts] + pbx[ts]
        y = pay[ts] + pby[ts]
        z = paz[ts] + pbz[ts]
        n2 = x * x + y * y + z * z
        n2c = jnp.maximum(n2, jnp.float32(1e-30))
        i = lax.bitcast_convert_type(n2c, jnp.int32)
        i = jnp.int32(0x5F3759DF) - lax.shift_right_arithmetic(i, jnp.int32(1))
        r = lax.bitcast_convert_type(i, jnp.float32)
        for _ in range(3):
            r = r * (jnp.float32(1.5) - jnp.float32(0.5) * n2c * r * r)
        norm = n2 * r
        inv = jnp.float32(1.0) / jnp.maximum(norm, jnp.float32(1e-6))
        ox[ts] = x * inv
        oy[ts] = y * inv
        oz[ts] = z * inv
        return ()

    lax.fori_loop(0, VN // L, step, (), unroll=False)
    pltpu.sync_copy(ox, out_hbm.at[pl.ds(0 * VP + base, VN)])
    pltpu.sync_copy(oy, out_hbm.at[pl.ds(1 * VP + base, VN)])
    pltpu.sync_copy(oz, out_hbm.at[pl.ds(2 * VP + base, VN)])


_mesh = plsc.VectorSubcoreMesh(core_axis_name="c", subcore_axis_name="s")

_accumulate = functools.partial(
    pl.kernel,
    mesh=_mesh,
    out_type=jax.ShapeDtypeStruct((NC * 3 * VP,), jnp.float32),
    scratch_types=(
        [pltpu.VMEM_SHARED((VP,), jnp.float32)] * 6
        + [pltpu.VMEM((G,), jnp.int32)] * 3
        + [pltpu.VMEM((G,), jnp.float32)] * 9
        + [pltpu.VMEM((G,), jnp.float32)] * 3
    ),
)(_accumulate_body)

_normalize = functools.partial(
    pl.kernel,
    mesh=_mesh,
    out_type=jax.ShapeDtypeStruct((3 * VP,), jnp.float32),
    scratch_types=[pltpu.VMEM((VN,), jnp.float32)] * 9,
)(_normalize_body)


def kernel(vertices, faces):
    vsoa = jnp.zeros((3, VP), jnp.float32).at[:, :V].set(vertices.T)
    idx = faces.astype(jnp.int32).T
    idx = jnp.pad(idx, ((0, 0), (0, NW * FW - F)))
    idx = idx.reshape(3, NW, GROUPS * G).transpose(1, 0, 2)
    zeros = jnp.zeros((VP,), jnp.float32)
    partials = _accumulate(vsoa.reshape(-1), idx.reshape(-1), zeros)
    out = _normalize(partials)
    return out.reshape(3, VP)[:, :V].T
